# baseline (speedup 1.0000x reference)
import functools

import jax
import jax.numpy as jnp
from jax import lax
from jax.experimental import pallas as pl
from jax.experimental.pallas import tpu as pltpu
from jax.experimental.pallas import tpu_sc as plsc

_NC = 2
_NS = 16
_NW = _NC * _NS

_B = 4
_S = 8192
_D = 1024
_ROWS = _B * _S
_RPW = _ROWS // _NW
_R = 16
_CHUNK = _R * _D
_NCHUNK = _RPW // _R


def _sc_body(x_hbm, emb_hbm, out_hbm, xbuf, ebuf):
    wid = lax.axis_index("s") * _NC + lax.axis_index("c")
    base_row = wid * _RPW
    ebase_row = base_row % _S

    def chunk_body(j, _):
        off = (base_row + j * _R) * _D
        eoff = (ebase_row + j * _R) * _D
        pltpu.sync_copy(x_hbm.at[pl.ds(off, _CHUNK)], xbuf)
        pltpu.sync_copy(emb_hbm.at[pl.ds(eoff, _CHUNK)], ebuf)

        def add_body(i, _):
            s = pl.ds(i * 16, 16)
            xbuf[s] = xbuf[s] + ebuf[s]
            return 0

        lax.fori_loop(0, _CHUNK // 16, add_body, 0)
        pltpu.sync_copy(xbuf, out_hbm.at[pl.ds(off, _CHUNK)])
        return 0

    lax.fori_loop(0, _NCHUNK, chunk_body, 0)


@functools.partial(
    pl.kernel,
    out_type=jax.ShapeDtypeStruct((_ROWS * _D,), jnp.float32),
    mesh=plsc.VectorSubcoreMesh(
        core_axis_name="c", subcore_axis_name="s", num_cores=_NC,
        num_subcores=_NS),
    scratch_types=[
        pltpu.VMEM((_CHUNK,), jnp.float32),
        pltpu.VMEM((_CHUNK,), jnp.float32),
    ],
)
def _sc_add(x_hbm, emb_hbm, out_hbm, xbuf, ebuf):
    _sc_body(x_hbm, emb_hbm, out_hbm, xbuf, ebuf)


def kernel(x, emb_table):
    batch, seq_len, d_model = x.shape
    out_flat = _sc_add(x.reshape(-1), emb_table.reshape(-1))
    return out_flat.reshape(batch, seq_len, d_model)


# speedup vs baseline: 1.8769x; 1.8769x over previous
import functools

import jax
import jax.numpy as jnp
from jax import lax
from jax.experimental import pallas as pl
from jax.experimental.pallas import tpu as pltpu
from jax.experimental.pallas import tpu_sc as plsc

_NC = 2
_NS = 16
_NW = _NC * _NS

_B = 4
_S = 8192
_D = 1024
_ROWS = _B * _S
_RPW = _ROWS // _NW
_R = 16
_CHUNK = _R * _D
_NCHUNK = _RPW // _R
_NB = 2


def _sc_body(x_hbm, emb_hbm, out_hbm, xbufs, ebufs, obufs, semx, seme, semo):
    wid = lax.axis_index("s") * _NC + lax.axis_index("c")
    base_row = wid * _RPW
    base = base_row * _D
    ebase = (base_row % _S) * _D
    last = (_NCHUNK - 1) * _CHUNK

    def start_in(b, j):
        off = lax.min(j * _CHUNK, last)
        pltpu.async_copy(x_hbm.at[pl.ds(base + off, _CHUNK)], xbufs[b],
                         semx.at[b])
        pltpu.async_copy(emb_hbm.at[pl.ds(ebase + off, _CHUNK)], ebufs[b],
                         seme.at[b])

    def wait_in(b):
        pltpu.make_async_copy(x_hbm.at[pl.ds(base, _CHUNK)], xbufs[b],
                              semx.at[b]).wait()
        pltpu.make_async_copy(emb_hbm.at[pl.ds(ebase, _CHUNK)], ebufs[b],
                              seme.at[b]).wait()

    def wait_out(b):
        pltpu.make_async_copy(obufs[b], out_hbm.at[pl.ds(base, _CHUNK)],
                              semo.at[b]).wait()

    for b in range(_NB):
        start_in(b, jnp.int32(b))

    @pl.loop(0, _NCHUNK, step=_NB)
    def _outer(j0):
        for b in range(_NB):
            j = j0 + b
            wait_in(b)

            @pl.when(j >= _NB)
            def _():
                wait_out(b)

            @plsc.parallel_loop(0, _CHUNK // 16, unroll=8)
            def _add(i):
                s = pl.ds(i * 16, 16)
                obufs[b][s] = xbufs[b][s] + ebufs[b][s]

            pltpu.async_copy(obufs[b], out_hbm.at[pl.ds(base + j * _CHUNK,
                                                        _CHUNK)], semo.at[b])
            start_in(b, j + _NB)

    for b in range(_NB):
        wait_in(b)
        wait_out(b)


@functools.partial(
    pl.kernel,
    out_type=jax.ShapeDtypeStruct((_ROWS * _D,), jnp.float32),
    mesh=plsc.VectorSubcoreMesh(
        core_axis_name="c", subcore_axis_name="s", num_cores=_NC,
        num_subcores=_NS),
    scratch_types=[
        [pltpu.VMEM((_CHUNK,), jnp.float32) for _ in range(_NB)],
        [pltpu.VMEM((_CHUNK,), jnp.float32) for _ in range(_NB)],
        [pltpu.VMEM((_CHUNK,), jnp.float32) for _ in range(_NB)],
        pltpu.SemaphoreType.DMA((_NB,)),
        pltpu.SemaphoreType.DMA((_NB,)),
        pltpu.SemaphoreType.DMA((_NB,)),
    ],
)
def _sc_add(x_hbm, emb_hbm, out_hbm, xbufs, ebufs, obufs, semx, seme, semo):
    _sc_body(x_hbm, emb_hbm, out_hbm, xbufs, ebufs, obufs, semx, seme, semo)


def kernel(x, emb_table):
    batch, seq_len, d_model = x.shape
    out_flat = _sc_add(x.reshape(-1), emb_table.reshape(-1))
    return out_flat.reshape(batch, seq_len, d_model)


# speedup vs baseline: 8.2259x; 4.3828x over previous
import jax
import jax.numpy as jnp
from jax.experimental import pallas as pl

_S_BLK = 1024


def _add_kernel(x_ref, emb_ref, out_ref):
    out_ref[...] = x_ref[...] + emb_ref[...]


def kernel(x, emb_table):
    batch, seq_len, d_model = x.shape
    xf = x.reshape(batch * seq_len, d_model)
    nsb = seq_len // _S_BLK
    out = pl.pallas_call(
        _add_kernel,
        grid=(nsb, batch),
        in_specs=[
            pl.BlockSpec((_S_BLK, d_model), lambda i, b: (b * nsb + i, 0)),
            pl.BlockSpec((_S_BLK, d_model), lambda i, b: (i, 0)),
        ],
        out_specs=pl.BlockSpec((_S_BLK, d_model), lambda i, b: (b * nsb + i, 0)),
        out_shape=jax.ShapeDtypeStruct(xf.shape, x.dtype),
    )(xf, emb_table)
    return out.reshape(batch, seq_len, d_model)


# speedup vs baseline: 8.5541x; 1.0399x over previous
import jax
import jax.numpy as jnp
from jax.experimental import pallas as pl

_S_BLK = 2048


def _add_kernel(x_ref, emb_ref, out_ref):
    out_ref[...] = x_ref[...] + emb_ref[...]


def kernel(x, emb_table):
    batch, seq_len, d_model = x.shape
    xf = x.reshape(batch * seq_len, d_model)
    nsb = seq_len // _S_BLK
    out = pl.pallas_call(
        _add_kernel,
        grid=(nsb, batch),
        in_specs=[
            pl.BlockSpec((_S_BLK, d_model), lambda i, b: (b * nsb + i, 0)),
            pl.BlockSpec((_S_BLK, d_model), lambda i, b: (i, 0)),
        ],
        out_specs=pl.BlockSpec((_S_BLK, d_model), lambda i, b: (b * nsb + i, 0)),
        out_shape=jax.ShapeDtypeStruct(xf.shape, x.dtype),
    )(xf, emb_table)
    return out.reshape(batch, seq_len, d_model)


# speedup vs baseline: 8.5677x; 1.0016x over previous
import jax
import jax.numpy as jnp
from jax.experimental import pallas as pl
from jax.experimental.pallas import tpu as pltpu

_S_BLK = 2048


def _add_kernel(x_ref, emb_ref, out_ref):
    out_ref[...] = x_ref[...] + emb_ref[...]


def kernel(x, emb_table):
    batch, seq_len, d_model = x.shape
    xf = x.reshape(batch * seq_len, d_model)
    nsb = seq_len // _S_BLK
    out = pl.pallas_call(
        _add_kernel,
        grid=(nsb, batch),
        in_specs=[
            pl.BlockSpec((_S_BLK, d_model), lambda i, b: (b * nsb + i, 0)),
            pl.BlockSpec((_S_BLK, d_model), lambda i, b: (i, 0)),
        ],
        out_specs=pl.BlockSpec((_S_BLK, d_model), lambda i, b: (b * nsb + i, 0)),
        out_shape=jax.ShapeDtypeStruct(xf.shape, x.dtype),
        compiler_params=pltpu.CompilerParams(
            vmem_limit_bytes=115 * 1024 * 1024),
    )(xf, emb_table)
    return out.reshape(batch, seq_len, d_model)
